# Initial kernel scaffold; baseline (speedup 1.0000x reference)
import jax
import jax.numpy as jnp
from jax.experimental import pallas as pl


def kernel(monosaccharides, table):
    raise NotImplementedError("write your pallas kernel here")


# speedup vs baseline: 2.3112x; 2.3112x over previous
import functools

import jax
import jax.numpy as jnp
from jax import lax
from jax.experimental import pallas as pl
from jax.experimental.pallas import tpu as pltpu
from jax.experimental.pallas import tpu_sc as plsc

BATCH = 16384
HIST = 200
EMBED = 32
VOCAB = 31
TOTAL = BATCH * HIST
NW = 32
PER_TILE = TOTAL // NW
CHUNK = 2048
NCHUNK = PER_TILE // CHUNK
LANES = 16


def _sc_embed(idx_flat, table_flat):
    mesh = plsc.VectorSubcoreMesh(core_axis_name="c", subcore_axis_name="s")

    @functools.partial(
        pl.kernel,
        mesh=mesh,
        out_type=jax.ShapeDtypeStruct((TOTAL * EMBED,), jnp.float32),
        scratch_types=[
            pltpu.VMEM((VOCAB * EMBED,), jnp.float32),
            pltpu.VMEM((CHUNK,), jnp.int32),
            pltpu.VMEM((CHUNK * EMBED,), jnp.float32),
        ],
        compiler_params=pltpu.CompilerParams(needs_layout_passes=False),
    )
    def k(idx_hbm, table_hbm, out_hbm, table_v, idx_v, rows_v):
        wid = lax.axis_index("s") * 2 + lax.axis_index("c")
        in_base = wid * PER_TILE
        out_base = in_base * EMBED
        pltpu.sync_copy(table_hbm, table_v)
        lane_off = lax.iota(jnp.int32, LANES) * EMBED

        def chunk_body(i, _):
            pltpu.sync_copy(idx_hbm.at[pl.ds(in_base + i * CHUNK, CHUNK)],
                            idx_v)

            def group_body(g, _):
                iv = idx_v[pl.ds(g * LANES, LANES)]
                rb = iv * EMBED
                ob = g * (LANES * EMBED) + lane_off
                for d in range(EMBED):
                    vals = plsc.load_gather(table_v, [rb + d])
                    plsc.store_scatter(rows_v, [ob + d], vals)
                return ()

            lax.fori_loop(0, CHUNK // LANES, group_body, ())
            pltpu.sync_copy(
                rows_v,
                out_hbm.at[pl.ds(out_base + i * CHUNK * EMBED, CHUNK * EMBED)])
            return ()

        lax.fori_loop(0, NCHUNK, chunk_body, ())

    return k(idx_flat, table_flat)


def kernel(monosaccharides, table):
    idx_flat = monosaccharides.reshape(TOTAL).astype(jnp.int32)
    out = _sc_embed(idx_flat, table.reshape(VOCAB * EMBED))
    return out.reshape(BATCH, HIST, EMBED)


# speedup vs baseline: 3.0108x; 1.3027x over previous
import functools

import jax
import jax.numpy as jnp
from jax import lax
from jax.experimental import pallas as pl
from jax.experimental.pallas import tpu as pltpu
from jax.experimental.pallas import tpu_sc as plsc

BATCH = 16384
HIST = 200
EMBED = 32
VOCAB = 31
TOTAL = BATCH * HIST
NW = 32
PER_TILE = TOTAL // NW
CHUNK = 2048
NCHUNK = PER_TILE // CHUNK
LANES = 16


def _sc_embed(idx_flat, table_flat):
    mesh = plsc.VectorSubcoreMesh(core_axis_name="c", subcore_axis_name="s")

    @functools.partial(
        pl.kernel,
        mesh=mesh,
        out_type=jax.ShapeDtypeStruct((TOTAL * EMBED,), jnp.float32),
        scratch_types=[
            pltpu.VMEM((VOCAB * EMBED,), jnp.float32),
            pltpu.VMEM((CHUNK,), jnp.int32),
            pltpu.VMEM((CHUNK * EMBED,), jnp.float32),
        ],
        compiler_params=pltpu.CompilerParams(needs_layout_passes=False),
    )
    def k(idx_hbm, table_hbm, out_hbm, table_v, idx_v, rows_v):
        wid = lax.axis_index("s") * 2 + lax.axis_index("c")
        in_base = wid * PER_TILE
        out_base = in_base * EMBED
        pltpu.sync_copy(table_hbm, table_v)
        lane_off = lax.iota(jnp.int32, LANES) * EMBED

        def chunk_body(i, _):
            pltpu.sync_copy(idx_hbm.at[pl.ds(in_base + i * CHUNK, CHUNK)],
                            idx_v)

            @plsc.parallel_loop(0, CHUNK // LANES, unroll=2)
            def group_body(g):
                iv = idx_v[pl.ds(g * LANES, LANES)]
                rb = iv * EMBED
                ob = g * (LANES * EMBED) + lane_off
                for d in range(EMBED):
                    vals = plsc.load_gather(table_v, [rb + d])
                    plsc.store_scatter(rows_v, [ob + d], vals)
            pltpu.sync_copy(
                rows_v,
                out_hbm.at[pl.ds(out_base + i * CHUNK * EMBED, CHUNK * EMBED)])
            return ()

        lax.fori_loop(0, NCHUNK, chunk_body, ())

    return k(idx_flat, table_flat)


def kernel(monosaccharides, table):
    idx_flat = monosaccharides.reshape(TOTAL).astype(jnp.int32)
    out = _sc_embed(idx_flat, table.reshape(VOCAB * EMBED))
    return out.reshape(BATCH, HIST, EMBED)


# speedup vs baseline: 6.2731x; 2.0836x over previous
import functools

import jax
import jax.numpy as jnp
from jax import lax
from jax.experimental import pallas as pl
from jax.experimental.pallas import tpu as pltpu
from jax.experimental.pallas import tpu_sc as plsc

BATCH = 16384
HIST = 200
EMBED = 32
VOCAB = 31
TOTAL = BATCH * HIST
NW = 32
PER_TILE = TOTAL // NW
CHUNK = 2048
NCHUNK = PER_TILE // CHUNK
LANES = 16


def _sc_embed(idx_flat, table_flat):
    mesh = plsc.VectorSubcoreMesh(core_axis_name="c", subcore_axis_name="s")

    @functools.partial(
        pl.kernel,
        mesh=mesh,
        out_type=jax.ShapeDtypeStruct((TOTAL * EMBED,), jnp.float32),
        scratch_types=[
            pltpu.VMEM((VOCAB * EMBED,), jnp.float32),
            pltpu.VMEM((CHUNK,), jnp.int32),
            pltpu.VMEM((CHUNK * EMBED,), jnp.float32),
        ],
        compiler_params=pltpu.CompilerParams(needs_layout_passes=False),
    )
    def k(idx_hbm, table_hbm, out_hbm, table_v, idx_v, rows_v):
        wid = lax.axis_index("s") * 2 + lax.axis_index("c")
        in_base = wid * PER_TILE
        out_base = in_base * EMBED
        pltpu.sync_copy(table_hbm, table_v)
        lane = lax.iota(jnp.int32, LANES)
        lane_off = lane * EMBED
        dskew = [(lane + t) & (EMBED - 1) for t in range(EMBED)]

        def chunk_body(i, _):
            pltpu.sync_copy(idx_hbm.at[pl.ds(in_base + i * CHUNK, CHUNK)],
                            idx_v)

            @plsc.parallel_loop(0, CHUNK // LANES, unroll=2)
            def group_body(g):
                iv = idx_v[pl.ds(g * LANES, LANES)]
                rb = iv * EMBED
                ob = g * (LANES * EMBED) + lane_off
                for t in range(EMBED):
                    vals = plsc.load_gather(table_v, [rb + dskew[t]])
                    plsc.store_scatter(rows_v, [ob + dskew[t]], vals)
            pltpu.sync_copy(
                rows_v,
                out_hbm.at[pl.ds(out_base + i * CHUNK * EMBED, CHUNK * EMBED)])
            return ()

        lax.fori_loop(0, NCHUNK, chunk_body, ())

    return k(idx_flat, table_flat)


def kernel(monosaccharides, table):
    idx_flat = monosaccharides.reshape(TOTAL).astype(jnp.int32)
    out = _sc_embed(idx_flat, table.reshape(VOCAB * EMBED))
    return out.reshape(BATCH, HIST, EMBED)
